# baseline (speedup 1.0000x reference)
import functools

import jax
import jax.numpy as jnp
from jax import lax
from jax.experimental import pallas as pl
from jax.experimental.pallas import tpu as pltpu
from jax.experimental.pallas import tpu_sc as plsc

_D = 32
_NW = 32
_K = 16
_LP = 64


def _emb_body(x_hbm, table_hbm, out_hbm, idx_stage, idx_pad, rows_a, rows_b,
              sem_g, sem_sa, sem_sb, *, L, ngroups, rows_per_w, stage_rows):
    wid = lax.axis_index("s") * 2 + lax.axis_index("c")
    row_base = wid * rows_per_w
    pltpu.sync_copy(x_hbm.at[pl.ds(wid * stage_rows, stage_rows)], idx_stage)

    lane = jax.lax.iota(jnp.int32, 16)

    def relayout(k, _):
        flat = k * 16 + lane
        v = plsc.load_gather(idx_stage, [flat >> 7, flat & 127])
        plsc.store_scatter(idx_pad, [flat // L, flat % L], v)
        return 0

    lax.fori_loop(0, rows_per_w * L // 16, relayout, 0)

    def gather(g, b, buf):
        r = g * _K + b
        return pltpu.make_async_copy(
            table_hbm.at[idx_pad.at[r]], buf.at[b], sem_g)

    def fire_gathers(g, buf):
        for b in range(_K):
            gather(g, b, buf).start()

    def drain_gathers(g, buf):
        for b in range(_K):
            gather(g, b, buf).wait()

    def store(g, buf, sem):
        return pltpu.make_async_copy(
            buf, out_hbm.at[pl.ds(row_base + g * _K, _K)], sem)

    fire_gathers(0, rows_a)

    def outer(h, _):
        ga = 2 * h
        gb = 2 * h + 1
        drain_gathers(ga, rows_a)

        @pl.when(h > 0)
        def _():
            store(gb - 2, rows_b, sem_sb).wait()
        fire_gathers(gb, rows_b)
        store(ga, rows_a, sem_sa).start()
        drain_gathers(gb, rows_b)
        store(ga, rows_a, sem_sa).wait()

        @pl.when(h < ngroups // 2 - 1)
        def _():
            fire_gathers(ga + 2, rows_a)
        store(gb, rows_b, sem_sb).start()
        return 0

    lax.fori_loop(0, ngroups // 2, outer, 0)
    store(ngroups - 1, rows_b, sem_sb).wait()


def kernel(x, table):
    B, L = x.shape
    rows_per_w = B // _NW
    ngroups = rows_per_w // _K
    total = B * L
    stage_rows = total // _NW // 128
    assert B % _NW == 0 and rows_per_w % _K == 0 and ngroups % 2 == 0
    assert total % (_NW * 128) == 0 and 4 * 16 >= L and _LP % 8 == 0

    mesh = plsc.VectorSubcoreMesh(core_axis_name="c", subcore_axis_name="s")
    body = functools.partial(_emb_body, L=L, ngroups=ngroups,
                             rows_per_w=rows_per_w, stage_rows=stage_rows)
    rows_t = pltpu.VMEM((_K, L, _D), jnp.float32)
    emb = pl.kernel(
        body,
        out_type=jax.ShapeDtypeStruct((B, L, _D), jnp.float32),
        mesh=mesh,
        scratch_types=[
            pltpu.VMEM((stage_rows, 128), jnp.int32),
            pltpu.VMEM((rows_per_w, L), jnp.int32),
            rows_t,
            rows_t,
            pltpu.SemaphoreType.DMA,
            pltpu.SemaphoreType.DMA,
            pltpu.SemaphoreType.DMA,
        ],
        compiler_params=pltpu.CompilerParams(use_tc_tiling_on_sc=False,
                                             needs_layout_passes=False),
    )
    return emb(x.reshape(total // 128, 128), table)
